# baseline (speedup 1.0000x reference)
import functools

import jax
import jax.numpy as jnp
from jax import lax
from jax.experimental import pallas as pl
from jax.experimental.pallas import tpu as pltpu
from jax.experimental.pallas import tpu_sc as plsc

_NUM_EMB = 1000000
_DIM = 32
_BATCH = 4096
_SEQ = 200
_L = 16

_info = plsc.get_sparse_core_info()
_NC, _NS = _info.num_cores, _info.num_subcores
_NW = _NC * _NS
_NBT = _BATCH // 128
_GS, _GBT = 8, 4
_SPG = _SEQ // _GS
_BTPG = _NBT // _GBT
_UNITS = _SPG * _BTPG
_NT = _SEQ * (_DIM // 8) * _NBT

_mesh = plsc.VectorSubcoreMesh(core_axis_name="c", subcore_axis_name="s")


@functools.partial(
    pl.kernel,
    mesh=_mesh,
    out_type=jax.ShapeDtypeStruct((_NT, 8, 128), jnp.float32),
    scratch_types=[
        pltpu.VMEM((_SPG, _BTPG, 128), jnp.int32),
        pltpu.VMEM((128, _DIM), jnp.float32),
        pltpu.VMEM((128, _DIM), jnp.float32),
        pltpu.VMEM((_DIM // 8, 8, 128), jnp.float32),
        pltpu.VMEM((_DIM // 8, 8, 128), jnp.float32),
        pltpu.SemaphoreType.DMA,
        pltpu.SemaphoreType.DMA,
        pltpu.SemaphoreType.DMA,
        pltpu.SemaphoreType.DMA,
    ],
    compiler_params=pltpu.CompilerParams(
        use_tc_tiling_on_sc=False, needs_layout_passes=False
    ),
)
def _gather_sc(table_hbm, idx_hbm, out_hbm, idx_v, rows0, rows1,
               tile0, tile1, semg0, semg1, sems0, sems1):
    wid = lax.axis_index("s") * _NC + lax.axis_index("c")
    gs = wid // _GBT
    gbt = wid % _GBT
    rows = (rows0, rows1)
    tile = (tile0, tile1)
    semg = (semg0, semg1)
    sems = (sems0, sems1)

    pltpu.sync_copy(
        idx_hbm.at[pl.ds(gs * _SPG, _SPG), pl.ds(gbt * _BTPG, _BTPG)], idx_v
    )

    iota = lax.iota(jnp.int32, _L)

    def fire_g(u, buf):
        pltpu.async_copy(
            table_hbm.at[idx_v.at[u // _BTPG, u % _BTPG]], rows[buf], semg[buf]
        )

    def drain_g(buf):
        pltpu.make_async_copy(
            table_hbm.at[pl.ds(0, 128)], rows[buf], semg[buf]
        ).wait()

    def transpose(buf):
        for dt in range(_DIM // 8):
            for sl in range(8):
                d = dt * 8 + sl
                col = jnp.full((_L,), d, jnp.int32)
                for lv in range(128 // _L):
                    row_idx = iota + (lv * _L)
                    vec = plsc.load_gather(rows[buf], [row_idx, col])
                    tile[buf][dt, sl, pl.ds(lv * _L, _L)] = vec

    def fire_s(u, buf):
        s = gs * _SPG + u // _BTPG
        bt = gbt * _BTPG + u % _BTPG
        for dt in range(_DIM // 8):
            t = (s * (_DIM // 8) + dt) * _NBT + bt
            pltpu.async_copy(tile[buf].at[dt], out_hbm.at[t], sems[buf])

    def drain_s(buf):
        pltpu.make_async_copy(tile[buf], out_hbm.at[pl.ds(0, _DIM // 8)],
                              sems[buf]).wait()

    fire_g(0, 0)
    drain_g(0)
    fire_g(1, 1)
    transpose(0)
    fire_s(0, 0)
    drain_g(1)
    fire_g(2, 0)
    transpose(1)
    fire_s(1, 1)

    @pl.loop(2, _UNITS - 2, step=2)
    def _steady(u0):
        for d_ in range(2):
            u = u0 + d_
            buf = d_ % 2
            nbuf = 1 - buf
            drain_s(buf)
            drain_g(buf)
            fire_g(u + 1, nbuf)
            transpose(buf)
            fire_s(u, buf)

    drain_s(0)
    drain_g(0)
    fire_g(_UNITS - 1, 1)
    transpose(0)
    fire_s(_UNITS - 2, 0)
    drain_s(1)
    drain_g(1)
    transpose(1)
    fire_s(_UNITS - 1, 1)
    drain_s(0)
    drain_s(1)


def kernel(input_ids, weight):
    idx3 = input_ids.T.reshape(_SEQ, _NBT, 128)
    out = _gather_sc(weight, idx3)
    res = out.reshape(_SEQ, _DIM // 8, _NBT, 8, 128)
    res = res.transpose(2, 4, 0, 1, 3)
    return res.reshape(_BATCH, _SEQ, _DIM)
